# baseline (speedup 1.0000x reference)
import functools

import jax
import jax.numpy as jnp
from jax import lax
from jax.experimental import pallas as pl
from jax.experimental.pallas import tpu as pltpu
from jax.experimental.pallas import tpu_sc as plsc

NUM_EMBEDDINGS = 1000000
D = 64
L = 16
DK = D // L
MAX_NORM = 1.0

NC = 2
NS = 16
NW = NC * NS

B_TOTAL = 4096 * 200
B_PER_W = B_TOTAL // NW
CHUNK = 512
N_CHUNKS = B_PER_W // CHUNK


def _rsqrt_newton(a):
    i = plsc.bitcast(a, jnp.int32)
    i = jnp.int32(0x5F3759DF) - (i >> 1)
    y = plsc.bitcast(i, jnp.float32)
    for _ in range(3):
        y = y * (jnp.float32(1.5) - jnp.float32(0.5) * a * y * y)
    return y


_mesh = plsc.VectorSubcoreMesh(core_axis_name="c", subcore_axis_name="s")


@functools.partial(
    pl.kernel,
    mesh=_mesh,
    out_type=jax.ShapeDtypeStruct((B_TOTAL, D), jnp.float32),
    scratch_types=[
        pltpu.VMEM((CHUNK,), jnp.int32),
        pltpu.VMEM((CHUNK,), jnp.int32),
        pltpu.VMEM((CHUNK, D), jnp.float32),
        pltpu.VMEM((CHUNK, D), jnp.float32),
        pltpu.VMEM((L, L), jnp.float32),
        pltpu.SemaphoreType.DMA,
        pltpu.SemaphoreType.DMA,
    ],
    compiler_params=pltpu.CompilerParams(needs_layout_passes=False,
                                         use_tc_tiling_on_sc=False),
)
def _gather_renorm(idx_hbm, table_hbm, out_hbm, idx_a, idx_b, rows_a, rows_b,
                   tbuf, sem_a, sem_b):
    wid = lax.axis_index("s") * NC + lax.axis_index("c")
    wbase = wid * B_PER_W
    lane = lax.iota(jnp.int32, L)

    def compute(buf):
        def grp_body(q, c):
            for rl in range(L):
                r = q * L + rl
                t = None
                for k in range(DK):
                    v = buf[r, pl.ds(k * L, L)]
                    t = v * v if t is None else t + v * v
                plsc.store_scatter(tbuf, [lane, jnp.full((L,), rl, jnp.int32)],
                                   t)
            a = None
            for i in range(L):
                row = tbuf[i, :]
                a = row if a is None else a + row
            y = _rsqrt_newton(a)
            scale16 = jnp.where(a > jnp.float32(MAX_NORM * MAX_NORM),
                                y * jnp.float32(MAX_NORM), jnp.float32(1.0))
            for rl in range(L):
                r = q * L + rl
                s = scale16[rl]
                for k in range(DK):
                    buf[r, pl.ds(k * L, L)] = buf[r, pl.ds(k * L, L)] * s
            return c

        lax.fori_loop(0, CHUNK // L, grp_body, 0)

    def pair_body(g2, carry):
        base_a = wbase + (2 * g2) * CHUNK
        base_b = base_a + CHUNK
        pltpu.sync_copy(idx_hbm.at[pl.ds(base_a, CHUNK)], idx_a)
        pltpu.sync_copy(idx_hbm.at[pl.ds(base_b, CHUNK)], idx_b)
        ha = pltpu.async_copy(table_hbm.at[idx_a], rows_a, sem_a)
        hb = pltpu.async_copy(table_hbm.at[idx_b], rows_b, sem_b)
        ha.wait()
        compute(rows_a)
        pltpu.sync_copy(rows_a, out_hbm.at[pl.ds(base_a, CHUNK)])
        hb.wait()
        compute(rows_b)
        pltpu.sync_copy(rows_b, out_hbm.at[pl.ds(base_b, CHUNK)])
        return carry

    lax.fori_loop(0, N_CHUNKS // 2, pair_body, 0)


def kernel(x, table):
    flat = x.reshape(-1).astype(jnp.int32)
    out = _gather_renorm(flat, table)
    return out.reshape(x.shape[0], x.shape[1], D)
